# baseline (speedup 1.0000x reference)
import functools

import jax
import jax.numpy as jnp
from jax import lax
from jax.experimental import pallas as pl
from jax.experimental.pallas import tpu as pltpu
from jax.experimental.pallas import tpu_sc as plsc

L = 16
NC, NS = 2, 16
NW = NC * NS

M, K, N = 16384, 4096, 64
P = (M * N) // NW
CHUNK = 128
NCH = P // CHUNK
GROUPS = CHUNK // L
ROWS_PER_CHUNK = CHUNK // N
ROWS_PER_W = P // N
NSL = N // L
DEPTH = 8

_mesh = plsc.VectorSubcoreMesh(core_axis_name="c", subcore_axis_name="s")


@functools.partial(
    pl.kernel,
    out_type=jax.ShapeDtypeStruct((M * N,), jnp.float32),
    mesh=_mesh,
    scratch_types=[
        pltpu.VMEM((N,), jnp.int32),
        pltpu.VMEM((NSL, L), jnp.int32),
        pltpu.VMEM((NCH, CHUNK), jnp.int32),
        pltpu.VMEM((P,), jnp.float32),
        pltpu.SemaphoreType.DMA,
    ],
)
def _sc_gather(xf, idxh, out, idxv, pre, idxbuf, stage, sem):
    wid = lax.axis_index("s") * NC + lax.axis_index("c")
    pltpu.sync_copy(idxh, idxv)
    for s in range(NSL):
        pre[s, :] = idxv[pl.ds(s * L, L)]
    base_row = wid * ROWS_PER_W

    def build_body(c, carry):
        for g in range(GROUPS):
            i = base_row + c * ROWS_PER_CHUNK + (g * L) // N
            s = (g * L) % N // L
            idxbuf[c, pl.ds(g * L, L)] = pre[s, :] + i * K
        return carry

    lax.fori_loop(0, NCH, build_body, 0)

    def fire(c):
        return pltpu.async_copy(
            xf.at[idxbuf.at[c]], stage.at[pl.ds(c * CHUNK, CHUNK)], sem)

    def drain_one():
        pltpu.make_async_copy(
            xf.at[idxbuf.at[0]], stage.at[pl.ds(0, CHUNK)], sem).wait()

    for b in range(DEPTH):
        fire(b)

    def dma_body(c, carry):
        drain_one()
        fire(c)
        return carry

    lax.fori_loop(DEPTH, NCH, dma_body, 0)
    for b in range(DEPTH):
        drain_one()

    pltpu.sync_copy(stage, out.at[pl.ds(wid * P, P)])


def kernel(x, idx):
    xf = x.reshape(M * K)
    return _sc_gather(xf, idx).reshape(M, N)


# speedup vs baseline: 3.1768x; 3.1768x over previous
import jax
import jax.numpy as jnp
from jax.experimental import pallas as pl

_BM = 512


def _body(idx_ref, x_ref, o_ref):
    idxv = idx_ref[...]
    cols = jax.lax.broadcasted_iota(jnp.int32, (4096, 64), 0)
    onehot = (cols == idxv).astype(jnp.float32)
    o_ref[...] = jnp.dot(x_ref[...], onehot, preferred_element_type=jnp.float32)


def kernel(x, idx):
    m, k = x.shape
    n = idx.shape[0]
    idx2 = idx.reshape(1, n)
    grid = (m // _BM,)
    return pl.pallas_call(
        _body,
        grid=grid,
        in_specs=[
            pl.BlockSpec((1, n), lambda i: (0, 0)),
            pl.BlockSpec((_BM, k), lambda i: (i, 0)),
        ],
        out_specs=pl.BlockSpec((_BM, n), lambda i: (i, 0)),
        out_shape=jax.ShapeDtypeStruct((m, n), jnp.float32),
    )(idx2, x)
